# Initial kernel scaffold; baseline (speedup 1.0000x reference)
import jax
import jax.numpy as jnp
from jax.experimental import pallas as pl


def kernel(patch_logits):
    raise NotImplementedError("write your pallas kernel here")


# speedup vs baseline: 4.4505x; 4.4505x over previous
import functools

import jax
import jax.numpy as jnp
from jax import lax
from jax.experimental import pallas as pl
from jax.experimental.pallas import tpu as pltpu
from jax.experimental.pallas import tpu_sc as plsc

B = 128
N = 32768
K = 3277
NBIN = 256
LANES = 16
NTILES = 32
ROWS_PER_TILE = B // NTILES
SHIFTS = (24, 16, 8, 0)

_mesh = plsc.VectorSubcoreMesh(core_axis_name="c", subcore_axis_name="s")


@functools.partial(
    pl.kernel,
    out_type=jax.ShapeDtypeStruct((NTILES, LANES), jnp.float32),
    mesh=_mesh,
    scratch_types=[
        pltpu.VMEM((N,), jnp.float32),
        pltpu.VMEM((NBIN * LANES,), jnp.int32),
        pltpu.VMEM((NBIN * LANES,), jnp.float32),
        pltpu.VMEM((LANES,), jnp.float32),
    ],
    compiler_params=pltpu.CompilerParams(needs_layout_passes=False),
)
def _topk_pool_sc(x_hbm, out_hbm, row_v, hist_v, vsum_v, out_v):
    wid = lax.axis_index("s") * 2 + lax.axis_index("c")
    iota = lax.broadcasted_iota(jnp.int32, (LANES,), 0)
    ones_i = jnp.ones((LANES,), jnp.int32)
    res = jnp.zeros((LANES,), jnp.float32)

    for i in range(ROWS_PER_TILE):
        r = wid * ROWS_PER_TILE + i
        pltpu.sync_copy(x_hbm.at[r], row_v)

        kr = jnp.int32(K)
        total = jnp.int32(N)
        pref = jnp.int32(0)
        sum_above = jnp.float32(0.0)

        for j, sh in enumerate(SHIFTS):
            def clr(c, _):
                hist_v[pl.ds(c * LANES, LANES)] = jnp.zeros((LANES,), jnp.int32)
                vsum_v[pl.ds(c * LANES, LANES)] = jnp.zeros((LANES,), jnp.float32)
                return 0

            lax.fori_loop(0, NBIN, clr, 0)

            prev_sh = SHIFTS[j - 1]
            pref_now = pref

            def hist_body(t, _):
                x = row_v[pl.ds(t * LANES, LANES)]
                bits = lax.bitcast_convert_type(x, jnp.int32)
                key = jnp.where(bits < 0, bits ^ jnp.int32(0x7FFFFFFF), bits)
                if j == 0:
                    digit = (key >> 24) + 128
                    m = None
                else:
                    digit = (key >> sh) & 0xFF
                    m = (key >> prev_sh) == pref_now
                idx = digit * LANES + iota
                plsc.addupdate_scatter(hist_v, [idx], ones_i, mask=m)
                plsc.addupdate_scatter(vsum_v, [idx], x, mask=m)
                return 0

            lax.fori_loop(0, N // LANES, hist_body, 0)

            cthr = total - kr

            def chunk(c, carry):
                run, runv, accb, accP, accPV, accT = carry
                base = c * (LANES * LANES) + iota * LANES
                t16 = jnp.zeros((LANES,), jnp.int32)
                v16 = jnp.zeros((LANES,), jnp.float32)
                for l in range(LANES):
                    t16 = t16 + plsc.load_gather(hist_v, [base + l])
                    v16 = v16 + plsc.load_gather(vsum_v, [base + l])
                P = run + plsc.cumsum(t16)
                PV = runv + plsc.cumsum(v16)
                sel = (P > cthr) & ((P - t16) <= cthr)
                binid = c * LANES + iota
                accb = jnp.where(sel, binid, accb)
                accP = jnp.where(sel, P, accP)
                accPV = jnp.where(sel, PV, accPV)
                accT = jnp.where(sel, t16, accT)
                run = run + jnp.sum(t16)
                runv = runv + jnp.sum(v16)
                return run, runv, accb, accP, accPV, accT

            z_i = jnp.zeros((LANES,), jnp.int32)
            z_f = jnp.zeros((LANES,), jnp.float32)
            run, runv, accb, accP, accPV, accT = lax.fori_loop(
                0, NBIN // LANES, chunk,
                (jnp.int32(0), jnp.float32(0.0), z_i, z_i, z_f, z_i))

            bstar = jnp.sum(accb)
            p_star = jnp.sum(accP)
            pv_star = jnp.sum(accPV)
            t_star = jnp.sum(accT)

            above = run - p_star
            vabove = runv - pv_star
            kr = kr - above
            sum_above = sum_above + vabove
            total = t_star
            if j == 0:
                pref = bstar - 128
            else:
                pref = (pref << 8) | bstar

        tbits = jnp.where(pref >= 0, pref, pref ^ jnp.int32(0x7FFFFFFF))
        tvec = jnp.zeros((LANES,), jnp.int32) + tbits
        vt = lax.bitcast_convert_type(tvec, jnp.float32)
        mean_vec = (sum_above + kr.astype(jnp.float32) * vt) * jnp.float32(1.0 / K)
        res = jnp.where(iota == i, mean_vec, res)

    out_v[...] = res
    pltpu.sync_copy(out_v, out_hbm.at[wid])


def kernel(patch_logits):
    out32 = _topk_pool_sc(patch_logits)
    return out32[:, :ROWS_PER_TILE].reshape(B, 1)


# speedup vs baseline: 17.3850x; 3.9063x over previous
import functools

import jax
import jax.numpy as jnp
from jax import lax
from jax.experimental import pallas as pl
from jax.experimental.pallas import tpu as pltpu
from jax.experimental.pallas import tpu_sc as plsc

B = 128
N = 32768
K = 3277
NBIN = 256
LANES = 16
NTILES = 32
ROWS_PER_TILE = B // NTILES
SHIFTS = (24, 16, 8, 0)

_mesh = plsc.VectorSubcoreMesh(core_axis_name="c", subcore_axis_name="s")


@functools.partial(
    pl.kernel,
    out_type=jax.ShapeDtypeStruct((NTILES, LANES), jnp.float32),
    mesh=_mesh,
    scratch_types=[
        pltpu.VMEM((N,), jnp.float32),
        pltpu.VMEM((N,), jnp.float32),
        pltpu.VMEM((NBIN * LANES,), jnp.int32),
        pltpu.VMEM((LANES,), jnp.float32),
        pltpu.SemaphoreType.DMA,
        pltpu.SemaphoreType.DMA,
    ],
    compiler_params=pltpu.CompilerParams(needs_layout_passes=False),
)
def _topk_pool_sc(x_hbm, out_hbm, row_a, row_b, hist_v, out_v, sem_a, sem_b):
    wid = lax.axis_index("s") * 2 + lax.axis_index("c")
    iota = lax.broadcasted_iota(jnp.int32, (LANES,), 0)
    ones_i = jnp.ones((LANES,), jnp.int32)
    res = jnp.zeros((LANES,), jnp.float32)

    rows = [row_a, row_b]
    sems = [sem_a, sem_b]
    copies = [None, None]
    copies[0] = pltpu.async_copy(
        x_hbm.at[wid * ROWS_PER_TILE], rows[0], sems[0])

    for i in range(ROWS_PER_TILE):
        cur = i % 2
        if i + 1 < ROWS_PER_TILE:
            copies[1 - cur] = pltpu.async_copy(
                x_hbm.at[wid * ROWS_PER_TILE + (i + 1)], rows[1 - cur],
                sems[1 - cur])
        copies[cur].wait()
        row_v = rows[cur]

        kr = jnp.int32(K)
        total = jnp.int32(N)
        pref = jnp.int32(0)

        for j, sh in enumerate(SHIFTS):
            @plsc.parallel_loop(0, NBIN, unroll=8)
            def _(c):
                hist_v[pl.ds(c * LANES, LANES)] = jnp.zeros((LANES,),
                                                            jnp.int32)

            prev_sh = SHIFTS[j - 1]
            pref_now = pref

            @plsc.parallel_loop(0, N // LANES, unroll=8)
            def _(t):
                x = row_v[pl.ds(t * LANES, LANES)]
                bits = lax.bitcast_convert_type(x, jnp.int32)
                key = jnp.where(bits < 0, bits ^ jnp.int32(0x7FFFFFFF), bits)
                if j == 0:
                    digit = (key >> 24) + 128
                    m = None
                else:
                    digit = (key >> sh) & 0xFF
                    m = (key >> prev_sh) == pref_now
                idx = digit * LANES + iota
                plsc.addupdate_scatter(hist_v, [idx], ones_i, mask=m)

            cthr = total - kr

            def chunk(c, carry):
                run, accb, accP, accT = carry
                base = c * (LANES * LANES) + iota * LANES
                t16 = jnp.zeros((LANES,), jnp.int32)
                for l in range(LANES):
                    t16 = t16 + plsc.load_gather(hist_v, [base + l])
                P = run + plsc.cumsum(t16)
                sel = (P > cthr) & ((P - t16) <= cthr)
                binid = c * LANES + iota
                accb = jnp.where(sel, binid, accb)
                accP = jnp.where(sel, P, accP)
                accT = jnp.where(sel, t16, accT)
                run = run + jnp.sum(t16)
                return run, accb, accP, accT

            z_i = jnp.zeros((LANES,), jnp.int32)
            run, accb, accP, accT = lax.fori_loop(
                0, NBIN // LANES, chunk, (jnp.int32(0), z_i, z_i, z_i))

            bstar = jnp.sum(accb)
            p_star = jnp.sum(accP)
            t_star = jnp.sum(accT)

            above = run - p_star
            kr = kr - above
            total = t_star
            if j == 0:
                pref = bstar - 128
            else:
                pref = (pref << 8) | bstar

        tkey = pref

        @plsc.parallel_loop(0, N // LANES, unroll=8,
                            carry=jnp.zeros((LANES,), jnp.float32))
        def sum_gt(t, acc):
            x = row_v[pl.ds(t * LANES, LANES)]
            bits = lax.bitcast_convert_type(x, jnp.int32)
            key = jnp.where(bits < 0, bits ^ jnp.int32(0x7FFFFFFF), bits)
            return acc + jnp.where(key > tkey, x, jnp.float32(0.0))

        tbits = jnp.where(tkey >= 0, tkey, tkey ^ jnp.int32(0x7FFFFFFF))
        tvec = jnp.zeros((LANES,), jnp.int32) + tbits
        vt = lax.bitcast_convert_type(tvec, jnp.float32)
        mean_vec = (jnp.sum(sum_gt) + kr.astype(jnp.float32) * vt) \
            * jnp.float32(1.0 / K)
        res = jnp.where(iota == i, mean_vec, res)

    out_v[...] = res
    pltpu.sync_copy(out_v, out_hbm.at[wid])


def kernel(patch_logits):
    out32 = _topk_pool_sc(patch_logits)
    return out32[:, :ROWS_PER_TILE].reshape(B, 1)


# speedup vs baseline: 19.4628x; 1.1195x over previous
import functools

import jax
import jax.numpy as jnp
from jax import lax
from jax.experimental import pallas as pl
from jax.experimental.pallas import tpu as pltpu
from jax.experimental.pallas import tpu_sc as plsc

B = 128
N = 32768
K = 3277
NBIN = 256
LANES = 16
NTILES = 32
ROWS_PER_TILE = B // NTILES
SHIFTS = (24, 16, 8, 0)

_mesh = plsc.VectorSubcoreMesh(core_axis_name="c", subcore_axis_name="s")


@functools.partial(
    pl.kernel,
    out_type=jax.ShapeDtypeStruct((NTILES, LANES), jnp.float32),
    mesh=_mesh,
    scratch_types=[
        pltpu.VMEM((N,), jnp.float32),
        pltpu.VMEM((N,), jnp.float32),
        pltpu.VMEM((N,), jnp.int32),
        pltpu.VMEM((NBIN * LANES,), jnp.int32),
        pltpu.VMEM((LANES,), jnp.float32),
        pltpu.SemaphoreType.DMA,
        pltpu.SemaphoreType.DMA,
    ],
    compiler_params=pltpu.CompilerParams(needs_layout_passes=False),
)
def _topk_pool_sc(x_hbm, out_hbm, row_a, row_b, keys_v, hist_v, out_v,
                  sem_a, sem_b):
    wid = lax.axis_index("s") * 2 + lax.axis_index("c")
    iota = lax.broadcasted_iota(jnp.int32, (LANES,), 0)
    ones_i = jnp.ones((LANES,), jnp.int32)
    res = jnp.zeros((LANES,), jnp.float32)

    rows = [row_a, row_b]
    sems = [sem_a, sem_b]
    copies = [None, None]
    copies[0] = pltpu.async_copy(
        x_hbm.at[wid * ROWS_PER_TILE], rows[0], sems[0])

    for i in range(ROWS_PER_TILE):
        cur = i % 2
        if i + 1 < ROWS_PER_TILE:
            copies[1 - cur] = pltpu.async_copy(
                x_hbm.at[wid * ROWS_PER_TILE + (i + 1)], rows[1 - cur],
                sems[1 - cur])
        copies[cur].wait()
        row_v = rows[cur]

        kr = jnp.int32(K)
        total = jnp.int32(N)
        pref = jnp.int32(0)

        for j, sh in enumerate(SHIFTS):
            @plsc.parallel_loop(0, NBIN, unroll=8)
            def _(c):
                hist_v[pl.ds(c * LANES, LANES)] = jnp.zeros((LANES,),
                                                            jnp.int32)

            prev_sh = SHIFTS[j - 1]
            pref_now = pref

            @plsc.parallel_loop(0, N // LANES, unroll=8)
            def _(t):
                if j == 0:
                    x = row_v[pl.ds(t * LANES, LANES)]
                    bits = lax.bitcast_convert_type(x, jnp.int32)
                    key = jnp.where(bits < 0, bits ^ jnp.int32(0x7FFFFFFF),
                                    bits)
                    keys_v[pl.ds(t * LANES, LANES)] = key
                    idx = ((key >> 24) + 128) * LANES + iota
                    m = None
                else:
                    key = keys_v[pl.ds(t * LANES, LANES)]
                    m = (key >> prev_sh) == pref_now
                    if sh == 0:
                        idx = ((key & 0xFF) << 4) | iota
                    else:
                        idx = (lax.shift_right_logical(key, sh - 4)
                               & 0xFF0) | iota
                plsc.addupdate_scatter(hist_v, [idx], ones_i, mask=m)

            cthr = total - kr

            def chunk(c, carry):
                run, accb, accP, accT = carry
                base = c * (LANES * LANES) + iota * LANES
                t16 = jnp.zeros((LANES,), jnp.int32)
                for l in range(LANES):
                    t16 = t16 + plsc.load_gather(hist_v, [base + l])
                P = run + plsc.cumsum(t16)
                sel = (P > cthr) & ((P - t16) <= cthr)
                binid = c * LANES + iota
                accb = jnp.where(sel, binid, accb)
                accP = jnp.where(sel, P, accP)
                accT = jnp.where(sel, t16, accT)
                run = run + jnp.sum(t16)
                return run, accb, accP, accT

            z_i = jnp.zeros((LANES,), jnp.int32)
            run, accb, accP, accT = lax.fori_loop(
                0, NBIN // LANES, chunk, (jnp.int32(0), z_i, z_i, z_i))

            bstar = jnp.sum(accb)
            p_star = jnp.sum(accP)
            t_star = jnp.sum(accT)

            above = run - p_star
            kr = kr - above
            total = t_star
            if j == 0:
                pref = bstar - 128
            else:
                pref = (pref << 8) | bstar

        tkey = pref
        tbits = jnp.where(tkey >= 0, tkey, tkey ^ jnp.int32(0x7FFFFFFF))
        tvec = jnp.zeros((LANES,), jnp.int32) + tbits
        vt = lax.bitcast_convert_type(tvec, jnp.float32)

        @plsc.parallel_loop(0, N // LANES, unroll=8,
                            carry=jnp.zeros((LANES,), jnp.float32))
        def sum_gt(t, acc):
            x = row_v[pl.ds(t * LANES, LANES)]
            return acc + jnp.where(x > vt, x, jnp.float32(0.0))
        mean_vec = (jnp.sum(sum_gt) + kr.astype(jnp.float32) * vt) \
            * jnp.float32(1.0 / K)
        res = jnp.where(iota == i, mean_vec, res)

    out_v[...] = res
    pltpu.sync_copy(out_v, out_hbm.at[wid])


def kernel(patch_logits):
    out32 = _topk_pool_sc(patch_logits)
    return out32[:, :ROWS_PER_TILE].reshape(B, 1)


# speedup vs baseline: 21.5547x; 1.1075x over previous
import functools

import jax
import jax.numpy as jnp
from jax import lax
from jax.experimental import pallas as pl
from jax.experimental.pallas import tpu as pltpu
from jax.experimental.pallas import tpu_sc as plsc

B = 128
N = 32768
K = 3277
NBIN = 256
LANES = 16
NTILES = 32
ROWS_PER_TILE = B // NTILES
SHIFTS = (24, 16, 8, 0)

_mesh = plsc.VectorSubcoreMesh(core_axis_name="c", subcore_axis_name="s")


@functools.partial(
    pl.kernel,
    out_type=jax.ShapeDtypeStruct((NTILES, LANES), jnp.float32),
    mesh=_mesh,
    scratch_types=[
        pltpu.VMEM((N,), jnp.float32),
        pltpu.VMEM((N,), jnp.float32),
        pltpu.VMEM((N,), jnp.int32),
        pltpu.VMEM((NBIN,), jnp.int32),
        pltpu.VMEM((LANES,), jnp.float32),
        pltpu.SemaphoreType.DMA,
        pltpu.SemaphoreType.DMA,
    ],
    compiler_params=pltpu.CompilerParams(needs_layout_passes=False),
)
def _topk_pool_sc(x_hbm, out_hbm, row_a, row_b, keys_v, hist_v, out_v,
                  sem_a, sem_b):
    wid = lax.axis_index("s") * 2 + lax.axis_index("c")
    iota = lax.broadcasted_iota(jnp.int32, (LANES,), 0)
    ones_i = jnp.ones((LANES,), jnp.int32)
    res = jnp.zeros((LANES,), jnp.float32)

    rows = [row_a, row_b]
    sems = [sem_a, sem_b]
    copies = [None, None]
    copies[0] = pltpu.async_copy(
        x_hbm.at[wid * ROWS_PER_TILE], rows[0], sems[0])

    for i in range(ROWS_PER_TILE):
        cur = i % 2
        if i + 1 < ROWS_PER_TILE:
            copies[1 - cur] = pltpu.async_copy(
                x_hbm.at[wid * ROWS_PER_TILE + (i + 1)], rows[1 - cur],
                sems[1 - cur])
        copies[cur].wait()
        row_v = rows[cur]

        kr = jnp.int32(K)
        total = jnp.int32(N)
        pref = jnp.int32(0)

        for j, sh in enumerate(SHIFTS):
            @plsc.parallel_loop(0, NBIN // LANES, unroll=4)
            def _(c):
                hist_v[pl.ds(c * LANES, LANES)] = jnp.zeros((LANES,),
                                                            jnp.int32)

            prev_sh = SHIFTS[j - 1]
            pref_now = pref

            @plsc.parallel_loop(0, N // LANES, unroll=8)
            def _(t):
                if j == 0:
                    x = row_v[pl.ds(t * LANES, LANES)]
                    bits = lax.bitcast_convert_type(x, jnp.int32)
                    key = jnp.where(bits < 0, bits ^ jnp.int32(0x7FFFFFFF),
                                    bits)
                    keys_v[pl.ds(t * LANES, LANES)] = key
                    digit = (key >> 24) + 128
                    m = None
                else:
                    key = keys_v[pl.ds(t * LANES, LANES)]
                    m = (key >> prev_sh) == pref_now
                    if sh == 0:
                        digit = key & 0xFF
                    else:
                        digit = (key >> sh) & 0xFF
                cnt, last = plsc.scan_count(digit, mask=m)
                plsc.addupdate_scatter(hist_v, [digit], cnt, mask=last)

            cthr = total - kr

            def chunk(c, carry):
                run, accb, accP, accT = carry
                t16 = hist_v[pl.ds(c * LANES, LANES)]
                P = run + plsc.cumsum(t16)
                sel = (P > cthr) & ((P - t16) <= cthr)
                binid = c * LANES + iota
                accb = jnp.where(sel, binid, accb)
                accP = jnp.where(sel, P, accP)
                accT = jnp.where(sel, t16, accT)
                run = run + jnp.sum(t16)
                return run, accb, accP, accT

            z_i = jnp.zeros((LANES,), jnp.int32)
            run, accb, accP, accT = lax.fori_loop(
                0, NBIN // LANES, chunk, (jnp.int32(0), z_i, z_i, z_i))

            bstar = jnp.sum(accb)
            p_star = jnp.sum(accP)
            t_star = jnp.sum(accT)

            above = run - p_star
            kr = kr - above
            total = t_star
            if j == 0:
                pref = bstar - 128
            else:
                pref = (pref << 8) | bstar

        tkey = pref
        tbits = jnp.where(tkey >= 0, tkey, tkey ^ jnp.int32(0x7FFFFFFF))
        tvec = jnp.zeros((LANES,), jnp.int32) + tbits
        vt = lax.bitcast_convert_type(tvec, jnp.float32)

        @plsc.parallel_loop(0, N // LANES, unroll=8,
                            carry=jnp.zeros((LANES,), jnp.float32))
        def sum_gt(t, acc):
            x = row_v[pl.ds(t * LANES, LANES)]
            return acc + jnp.where(x > vt, x, jnp.float32(0.0))
        mean_vec = (jnp.sum(sum_gt) + kr.astype(jnp.float32) * vt) \
            * jnp.float32(1.0 / K)
        res = jnp.where(iota == i, mean_vec, res)

    out_v[...] = res
    pltpu.sync_copy(out_v, out_hbm.at[wid])


def kernel(patch_logits):
    out32 = _topk_pool_sc(patch_logits)
    return out32[:, :ROWS_PER_TILE].reshape(B, 1)
